# baseline (speedup 1.0000x reference)
import math

import jax
import jax.numpy as jnp
from jax import lax
from jax.experimental import pallas as pl
from jax.experimental.pallas import tpu as pltpu
from jax.experimental.pallas import tpu_sc as plsc

EMB = 64
SCALE = math.sqrt(EMB)
B_TOK = 4096 * 50
NC, NS, L = 2, 16, 16
NW = NC * NS
N_PER_W = B_TOK // NW
G = 128
IDX_ROWS = N_PER_W // G
K = 5
CHUNK = K * G
NCH = N_PER_W // CHUNK
SL = EMB // L


def _emb_body(tok_hbm, tbl_hbm, out_hbm, idx_v, buf, gsem):
    wid = lax.axis_index("s") * NC + lax.axis_index("c")
    base = wid * N_PER_W
    pltpu.sync_copy(tok_hbm.at[wid], idx_v)

    def chunk_body(g, carry):
        cps = [
            pltpu.async_copy(
                tbl_hbm.at[idx_v.at[g * K + j]],
                buf.at[pl.ds(j * G, G)],
                gsem,
            )
            for j in range(K)
        ]
        for cp in cps:
            cp.wait()

        def mul_row(r, c):
            for s in range(SL):
                buf[r, s, :] = buf[r, s, :] * SCALE
            return c

        lax.fori_loop(0, CHUNK, mul_row, 0, unroll=2)

        pltpu.sync_copy(buf, out_hbm.at[pl.ds(base + g * CHUNK, CHUNK)])
        return carry

    lax.fori_loop(0, NCH, chunk_body, 0)


@jax.jit
def _emb_call(tok3, tbl3):
    mesh = plsc.VectorSubcoreMesh(core_axis_name="c", subcore_axis_name="s")
    return pl.kernel(
        _emb_body,
        mesh=mesh,
        compiler_params=pltpu.CompilerParams(use_tc_tiling_on_sc=False),
        out_type=jax.ShapeDtypeStruct((B_TOK, SL, L), jnp.float32),
        scratch_types=[
            pltpu.VMEM((IDX_ROWS, G), jnp.int32),
            pltpu.VMEM((CHUNK, SL, L), jnp.float32),
            pltpu.SemaphoreType.DMA,
        ],
    )(tok3, tbl3)


def kernel(tokens, table):
    tok3 = tokens.astype(jnp.int32).reshape(NW, IDX_ROWS, G)
    tbl3 = table.reshape(1000000, SL, L)
    out = _emb_call(tok3, tbl3)
    return out.reshape(tokens.shape[0], tokens.shape[1], EMB)


# speedup vs baseline: 2.5781x; 2.5781x over previous
import math

import jax
import jax.numpy as jnp
from jax import lax
from jax.experimental import pallas as pl
from jax.experimental.pallas import tpu as pltpu
from jax.experimental.pallas import tpu_sc as plsc

EMB = 64
SCALE = math.sqrt(EMB)
B_TOK = 4096 * 50
NC, NS, L = 2, 16, 16
NW = NC * NS
N_PER_W = B_TOK // NW
G = 128
IDX_ROWS = N_PER_W // G
K = 5
CHUNK = K * G
NCH = N_PER_W // CHUNK
BB = 128


def _gather_body(tok_hbm, tbl_hbm, out_hbm, idx_v, buf, gsem):
    wid = lax.axis_index("s") * NC + lax.axis_index("c")
    base = wid * N_PER_W
    pltpu.sync_copy(tok_hbm.at[wid], idx_v)

    def chunk_body(g, carry):
        cps = [
            pltpu.async_copy(
                tbl_hbm.at[idx_v.at[g * K + j]],
                buf.at[pl.ds(j * G, G)],
                gsem,
            )
            for j in range(K)
        ]
        for cp in cps:
            cp.wait()
        pltpu.sync_copy(buf, out_hbm.at[pl.ds(base + g * CHUNK, CHUNK)])
        return carry

    lax.fori_loop(0, NCH, chunk_body, 0)


@jax.jit
def _emb_call(tok3, tbl_pad):
    mesh = plsc.VectorSubcoreMesh(core_axis_name="c", subcore_axis_name="s")
    return pl.kernel(
        _gather_body,
        mesh=mesh,
        compiler_params=pltpu.CompilerParams(use_tc_tiling_on_sc=False),
        out_type=jax.ShapeDtypeStruct((B_TOK, 2 * EMB), jnp.float32),
        scratch_types=[
            pltpu.VMEM((IDX_ROWS, G), jnp.int32),
            pltpu.VMEM((CHUNK, 2 * EMB), jnp.float32),
            pltpu.SemaphoreType.DMA,
        ],
    )(tok3, tbl_pad)


def _finish_body(rows_ref, out_ref):
    rows = rows_ref[...]
    out_ref[...] = rows[:, :EMB].reshape(BB, 50, EMB) * SCALE


@jax.jit
def _finish_call(rows):
    return pl.pallas_call(
        _finish_body,
        grid=(4096 // BB,),
        in_specs=[pl.BlockSpec((BB * 50, 2 * EMB), lambda i: (i, 0))],
        out_specs=pl.BlockSpec((BB, 50, EMB), lambda i: (i, 0, 0)),
        out_shape=jax.ShapeDtypeStruct((4096, 50, EMB), jnp.float32),
    )(rows)


def kernel(tokens, table):
    tok3 = tokens.astype(jnp.int32).reshape(NW, IDX_ROWS, G)
    tbl_pad = jnp.pad(table, ((0, 0), (0, 2 * EMB - EMB)))
    rows = _emb_call(tok3, tbl_pad)
    return _finish_call(rows)


# speedup vs baseline: 3.3872x; 1.3138x over previous
import math

import jax
import jax.numpy as jnp
from jax import lax
from jax.experimental import pallas as pl
from jax.experimental.pallas import tpu as pltpu
from jax.experimental.pallas import tpu_sc as plsc

EMB = 64
SCALE = math.sqrt(EMB)
B_TOK = 4096 * 50
HALF_V = 500000
NC, NS, L = 2, 16, 16
NW = NC * NS
N_PER_W = B_TOK // NW
G = 128
IDX_ROWS = N_PER_W // G
K = 5
CHUNK = K * G
NCH = N_PER_W // CHUNK
FC = 4096
BB = 128


def _fmt_body(in_t, out_ref):
    out_ref[:, :EMB] = jnp.transpose(in_t[...])
    out_ref[:, EMB:] = jnp.zeros((FC, EMB), jnp.float32)


@jax.jit
def _fmt_call(table_t):
    return pl.pallas_call(
        _fmt_body,
        grid=((1000000 + FC - 1) // FC,),
        in_specs=[pl.BlockSpec((EMB, FC), lambda i: (0, i))],
        out_specs=pl.BlockSpec((FC, 2 * EMB), lambda i: (i, 0)),
        out_shape=jax.ShapeDtypeStruct((1000000, 2 * EMB), jnp.float32),
    )(table_t)


def _gather_body(tok_hbm, tbl_hbm, out_hbm, idx_v, buf, gsem):
    wid = lax.axis_index("s") * NC + lax.axis_index("c")
    base = wid * N_PER_W
    pltpu.sync_copy(tok_hbm.at[wid], idx_v)

    def chunk_body(g, carry):
        cps = [
            pltpu.async_copy(
                tbl_hbm.at[idx_v.at[g * K + j]],
                buf.at[pl.ds(j * G, G)],
                gsem,
            )
            for j in range(K)
        ]
        for cp in cps:
            cp.wait()
        pltpu.sync_copy(buf, out_hbm.at[pl.ds(base + g * CHUNK, CHUNK)])
        return carry

    lax.fori_loop(0, NCH, chunk_body, 0)


@jax.jit
def _emb_call(tok3, tbl2):
    mesh = plsc.VectorSubcoreMesh(core_axis_name="c", subcore_axis_name="s")
    return pl.kernel(
        _gather_body,
        mesh=mesh,
        compiler_params=pltpu.CompilerParams(use_tc_tiling_on_sc=False),
        out_type=jax.ShapeDtypeStruct((B_TOK, 2 * EMB), jnp.float32),
        scratch_types=[
            pltpu.VMEM((IDX_ROWS, G), jnp.int32),
            pltpu.VMEM((CHUNK, 2 * EMB), jnp.float32),
            pltpu.SemaphoreType.DMA,
        ],
    )(tok3, tbl2)


def _finish_body(rows_ref, out_ref):
    rows = rows_ref[...]
    out_ref[...] = rows[:, :EMB].reshape(BB, 50, EMB) * SCALE


@jax.jit
def _finish_call(rows):
    return pl.pallas_call(
        _finish_body,
        grid=(4096 // BB,),
        in_specs=[pl.BlockSpec((BB * 50, 2 * EMB), lambda i: (i, 0))],
        out_specs=pl.BlockSpec((BB, 50, EMB), lambda i: (i, 0, 0)),
        out_shape=jax.ShapeDtypeStruct((4096, 50, EMB), jnp.float32),
    )(rows)


def kernel(tokens, table):
    tok3 = tokens.astype(jnp.int32).reshape(NW, IDX_ROWS, G)
    tbl2 = _fmt_call(table.T)
    rows = _emb_call(tok3, tbl2)
    return _finish_call(rows)


# speedup vs baseline: 3.8049x; 1.1233x over previous
import math

import jax
import jax.numpy as jnp
from jax import lax
from jax.experimental import pallas as pl
from jax.experimental.pallas import tpu as pltpu
from jax.experimental.pallas import tpu_sc as plsc

EMB = 64
SCALE = math.sqrt(EMB)
B_TOK = 4096 * 50
HALF_V = 512000
NC, NS, L = 2, 16, 16
NW = NC * NS
N_PER_W = B_TOK // NW
G = 128
IDX_ROWS = N_PER_W // G
K = 5
CHUNK = K * G
NCH = N_PER_W // CHUNK
FC = 4096
BB = 128


def _fmt_body(in_l, in_r, out_ref):
    out_ref[:, :EMB] = jnp.transpose(in_l[...])
    out_ref[:, EMB:] = jnp.transpose(in_r[...])


@jax.jit
def _fmt_call(table_t):
    return pl.pallas_call(
        _fmt_body,
        grid=(HALF_V // FC,),
        in_specs=[
            pl.BlockSpec((EMB, FC), lambda i: (0, i)),
            pl.BlockSpec(
                (EMB, FC),
                lambda i: (0, jnp.minimum(i + HALF_V // FC, 1000000 // FC)),
            ),
        ],
        out_specs=pl.BlockSpec((FC, 2 * EMB), lambda i: (i, 0)),
        out_shape=jax.ShapeDtypeStruct((HALF_V, 2 * EMB), jnp.float32),
    )(table_t, table_t)


def _gather_body(tok_hbm, tbl_hbm, out_hbm, idx_v, buf, gsem):
    wid = lax.axis_index("s") * NC + lax.axis_index("c")
    base = wid * N_PER_W
    pltpu.sync_copy(tok_hbm.at[wid], idx_v)

    def chunk_body(g, carry):
        cps = [
            pltpu.async_copy(
                tbl_hbm.at[idx_v.at[g * K + j]],
                buf.at[pl.ds(j * G, G)],
                gsem,
            )
            for j in range(K)
        ]
        for cp in cps:
            cp.wait()
        pltpu.sync_copy(buf, out_hbm.at[pl.ds(base + g * CHUNK, CHUNK)])
        return carry

    lax.fori_loop(0, NCH, chunk_body, 0)


@jax.jit
def _emb_call(tok3, tbl2):
    mesh = plsc.VectorSubcoreMesh(core_axis_name="c", subcore_axis_name="s")
    return pl.kernel(
        _gather_body,
        mesh=mesh,
        compiler_params=pltpu.CompilerParams(use_tc_tiling_on_sc=False),
        out_type=jax.ShapeDtypeStruct((B_TOK, 2 * EMB), jnp.float32),
        scratch_types=[
            pltpu.VMEM((IDX_ROWS, G), jnp.int32),
            pltpu.VMEM((CHUNK, 2 * EMB), jnp.float32),
            pltpu.SemaphoreType.DMA,
        ],
    )(tok3, tbl2)


def _finish_body(rows_ref, par_ref, out_ref):
    rows = rows_ref[...]
    a = rows[:, :EMB].reshape(BB, 50, EMB)
    b = rows[:, EMB:].reshape(BB, 50, EMB)
    p = par_ref[...].reshape(BB, 50, 1)
    out_ref[...] = jnp.where(p == 0, a, b) * SCALE


@jax.jit
def _finish_call(rows, par):
    return pl.pallas_call(
        _finish_body,
        grid=(4096 // BB,),
        in_specs=[
            pl.BlockSpec((BB * 50, 2 * EMB), lambda i: (i, 0)),
            pl.BlockSpec((BB, 50), lambda i: (i, 0)),
        ],
        out_specs=pl.BlockSpec((BB, 50, EMB), lambda i: (i, 0, 0)),
        out_shape=jax.ShapeDtypeStruct((4096, 50, EMB), jnp.float32),
    )(rows, par)


def kernel(tokens, table):
    tok = tokens.astype(jnp.int32)
    par = (tok >= HALF_V).astype(jnp.int32)
    tok3 = (tok - par * HALF_V).reshape(NW, IDX_ROWS, G)
    tbl2 = _fmt_call(table.T)
    rows = _emb_call(tok3, tbl2)
    return _finish_call(rows, par)


# speedup vs baseline: 3.9655x; 1.0422x over previous
import math

import jax
import jax.numpy as jnp
from jax import lax
from jax.experimental import pallas as pl
from jax.experimental.pallas import tpu as pltpu
from jax.experimental.pallas import tpu_sc as plsc

EMB = 64
SCALE = math.sqrt(EMB)
B_TOK = 4096 * 50
HALF_V = 512000
NC, NS, L = 2, 16, 16
NW = NC * NS
N_PER_W = B_TOK // NW
G = 128
IDX_ROWS = N_PER_W // G
K = 5
CHUNK = K * G
NCH = N_PER_W // CHUNK
FC = 4096
BB = 128


def _fmt_body(in_l, in_r, out_ref):
    out_ref[:, :EMB] = jnp.transpose(in_l[...])
    out_ref[:, EMB:] = jnp.transpose(in_r[...])


@jax.jit
def _fmt_call(table_t):
    return pl.pallas_call(
        _fmt_body,
        grid=(HALF_V // FC,),
        in_specs=[
            pl.BlockSpec((EMB, FC), lambda i: (0, i)),
            pl.BlockSpec(
                (EMB, FC),
                lambda i: (0, jnp.minimum(i + HALF_V // FC, 1000000 // FC)),
            ),
        ],
        out_specs=pl.BlockSpec((FC, 2 * EMB), lambda i: (i, 0)),
        out_shape=jax.ShapeDtypeStruct((HALF_V, 2 * EMB), jnp.float32),
    )(table_t, table_t)


def _gather_body(tok_hbm, tbl_hbm, out_hbm, idx_v, buf, gsem):
    wid = lax.axis_index("s") * NC + lax.axis_index("c")
    base = wid * N_PER_W
    pltpu.sync_copy(tok_hbm.at[wid], idx_v)

    def chunk_body(g, carry):
        cps = [
            pltpu.async_copy(
                tbl_hbm.at[idx_v.at[g * K + j]],
                buf.at[pl.ds(j * G, G)],
                gsem,
            )
            for j in range(K)
        ]
        for cp in cps:
            cp.wait()
        pltpu.sync_copy(buf, out_hbm.at[pl.ds(base + g * CHUNK, CHUNK)])
        return carry

    lax.fori_loop(0, NCH, chunk_body, 0)


@jax.jit
def _emb_call(tok3, tbl2):
    mesh = plsc.VectorSubcoreMesh(core_axis_name="c", subcore_axis_name="s")
    return pl.kernel(
        _gather_body,
        mesh=mesh,
        compiler_params=pltpu.CompilerParams(use_tc_tiling_on_sc=False),
        out_type=jax.ShapeDtypeStruct((B_TOK, 2 * EMB), jnp.float32),
        scratch_types=[
            pltpu.VMEM((IDX_ROWS, G), jnp.int32),
            pltpu.VMEM((CHUNK, 2 * EMB), jnp.float32),
            pltpu.SemaphoreType.DMA,
        ],
    )(tok3, tbl2)


def _finish_body(rows_ref, par_ref, out_ref):
    rows = rows_ref[...]
    a = rows[:, :EMB].reshape(BB, 50, EMB)
    b = rows[:, EMB:].reshape(BB, 50, EMB)
    p = par_ref[...].reshape(BB, 50, 1)
    sel = jnp.where(p == 0, a, b) * SCALE
    out_ref[...] = jnp.transpose(sel, (1, 2, 0))


@jax.jit
def _finish_call(rows, par):
    return pl.pallas_call(
        _finish_body,
        grid=(4096 // BB,),
        in_specs=[
            pl.BlockSpec((BB * 50, 2 * EMB), lambda i: (i, 0)),
            pl.BlockSpec((BB, 50), lambda i: (i, 0)),
        ],
        out_specs=pl.BlockSpec((50, EMB, BB), lambda i: (0, 0, i)),
        out_shape=jax.ShapeDtypeStruct((50, EMB, 4096), jnp.float32),
    )(rows, par)


def kernel(tokens, table):
    tok = tokens.astype(jnp.int32)
    par = (tok >= HALF_V).astype(jnp.int32)
    tok3 = (tok - par * HALF_V).reshape(NW, IDX_ROWS, G)
    tbl2 = _fmt_call(table.T)
    rows = _emb_call(tok3, tbl2)
    return jnp.transpose(_finish_call(rows, par), (2, 0, 1))
